# baseline (speedup 1.0000x reference)
import functools

import jax
import jax.numpy as jnp
from jax import lax
from jax.experimental import pallas as pl
from jax.experimental.pallas import tpu as pltpu
from jax.experimental.pallas import tpu_sc as plsc

N_NODES = 10000
N_EDGES = 320000
D = 128

NC = 2
NS = 16
NW = NC * NS
CHUNK = 56
NCH = 180
EPW = NCH * CHUNK
N_ACC = N_NODES + 64
ROWS_PER_SUB = 624
TAIL_BASE = NS * ROWS_PER_SUB
TAIL = N_NODES - TAIL_BASE

_mesh = plsc.VectorSubcoreMesh(core_axis_name="c", subcore_axis_name="s")


@functools.partial(
    pl.kernel,
    mesh=_mesh,
    out_type=jax.ShapeDtypeStruct((2, N_NODES, D), jnp.float32),
    scratch_types=[
        pltpu.VMEM((EPW,), jnp.int32),
        pltpu.VMEM((EPW,), jnp.int32),
        pltpu.VMEM((CHUNK, D), jnp.float32),
        pltpu.VMEM((CHUNK, D), jnp.float32),
        pltpu.VMEM((CHUNK, D), jnp.float32),
        pltpu.VMEM((CHUNK, D), jnp.float32),
        pltpu.VMEM_SHARED((N_ACC, D), jnp.float32),
        pltpu.SemaphoreType.DMA,
        pltpu.SemaphoreType.DMA,
        pltpu.SemaphoreType.DMA,
        pltpu.SemaphoreType.DMA,
    ],
)
def _edge_agg(src_hbm, dst_hbm, h_hbm, out_hbm, sidx, didx, rows0, rows1,
              rows2, rows3, acc, semg0, semg1, semg2, semg3):
    cid = lax.axis_index("c")
    sid = lax.axis_index("s")
    wid = cid * NS + sid
    ebase = wid * EPW

    rowsb = (rows0, rows1, rows2, rows3)
    semg = (semg0, semg1, semg2, semg3)

    def _chunk(ref, i):
        return ref.at[pl.ds(pl.multiple_of(i * CHUNK, 8), CHUNK)]

    def _g_fire(i, b):
        pltpu.async_copy(h_hbm.at[_chunk(sidx, i)], rowsb[b], semg[b])

    def _g_wait(b):
        pltpu.make_async_copy(h_hbm.at[_chunk(sidx, 0)], rowsb[b],
                              semg[b]).wait()

    base = sid * ROWS_PER_SUB
    pltpu.async_copy(src_hbm.at[pl.ds(ebase, EPW)], sidx, semg0)
    pltpu.async_copy(dst_hbm.at[pl.ds(ebase, EPW)], didx, semg1)
    pltpu.async_copy(h_hbm.at[pl.ds(base, ROWS_PER_SUB)],
                     acc.at[pl.ds(base, ROWS_PER_SUB)], semg2)
    pltpu.make_async_copy(src_hbm.at[pl.ds(ebase, EPW)], sidx,
                          semg0).wait()
    _g_fire(0, 0)

    @pl.when(sid == NS - 1)
    def _():
        pltpu.sync_copy(h_hbm.at[pl.ds(TAIL_BASE, TAIL)],
                        acc.at[pl.ds(TAIL_BASE, TAIL)])

    pltpu.make_async_copy(dst_hbm.at[pl.ds(ebase, EPW)], didx,
                          semg1).wait()
    _g_fire(1, 1)
    pltpu.make_async_copy(h_hbm.at[pl.ds(base, ROWS_PER_SUB)],
                          acc.at[pl.ds(base, ROWS_PER_SUB)], semg2).wait()
    _g_fire(2, 2)
    _g_fire(3, 3)

    plsc.subcore_barrier()

    def body(j, carry):
        c0 = 4 * j
        for k in range(4):
            i = c0 + k
            _g_wait(k)
            pltpu.sync_copy(rowsb[k], acc.at[_chunk(didx, i)], add=True)

            @pl.when(i + 4 < NCH)
            def _():
                _g_fire(i + 4, k)

        return carry

    lax.fori_loop(0, NCH // 4, body, 0)

    plsc.subcore_barrier()
    pltpu.sync_copy(acc.at[pl.ds(base, ROWS_PER_SUB)],
                    out_hbm.at[cid, pl.ds(base, ROWS_PER_SUB)])

    @pl.when(sid == NS - 1)
    def _():
        pltpu.sync_copy(acc.at[pl.ds(TAIL_BASE, TAIL)],
                        out_hbm.at[cid, pl.ds(TAIL_BASE, TAIL)])


_BLK = 1000
_GRID = N_NODES // _BLK


def _mm(x, W, b):
    def body(x_ref, w_ref, b_ref, o_ref):
        o_ref[...] = jnp.dot(x_ref[...], w_ref[...],
                             preferred_element_type=jnp.float32) + b_ref[...]

    return pl.pallas_call(
        body,
        grid=(_GRID,),
        in_specs=[pl.BlockSpec((_BLK, D), lambda i: (i, 0)),
                  pl.BlockSpec((D, D), lambda i: (0, 0)),
                  pl.BlockSpec((1, D), lambda i: (0, 0))],
        out_specs=pl.BlockSpec((_BLK, D), lambda i: (i, 0)),
        out_shape=jax.ShapeDtypeStruct((N_NODES, D), jnp.float32),
    )(x, W, b.reshape(1, D))


def _combine_mm(p0, p1, h, W, b):
    def body(p0_ref, p1_ref, h_ref, w_ref, b_ref, o_ref):
        z = jnp.maximum(p0_ref[...] + p1_ref[...] - h_ref[...], 0.0)
        o_ref[...] = jnp.dot(z, w_ref[...],
                             preferred_element_type=jnp.float32) + b_ref[...]

    return pl.pallas_call(
        body,
        grid=(_GRID,),
        in_specs=[pl.BlockSpec((_BLK, D), lambda i: (i, 0)),
                  pl.BlockSpec((_BLK, D), lambda i: (i, 0)),
                  pl.BlockSpec((_BLK, D), lambda i: (i, 0)),
                  pl.BlockSpec((D, D), lambda i: (0, 0)),
                  pl.BlockSpec((1, D), lambda i: (0, 0))],
        out_specs=pl.BlockSpec((_BLK, D), lambda i: (i, 0)),
        out_shape=jax.ShapeDtypeStruct((N_NODES, D), jnp.float32),
    )(p0, p1, h, W, b.reshape(1, D))


def _combine_relu(p0, p1, h):
    def body(p0_ref, p1_ref, h_ref, o_ref):
        o_ref[...] = jnp.maximum(p0_ref[...] + p1_ref[...] - h_ref[...], 0.0)

    return pl.pallas_call(
        body,
        grid=(_GRID,),
        in_specs=[pl.BlockSpec((_BLK, D), lambda i: (i, 0)),
                  pl.BlockSpec((_BLK, D), lambda i: (i, 0)),
                  pl.BlockSpec((_BLK, D), lambda i: (i, 0))],
        out_specs=pl.BlockSpec((_BLK, D), lambda i: (i, 0)),
        out_shape=jax.ShapeDtypeStruct((N_NODES, D), jnp.float32),
    )(p0, p1, h)


def kernel(edge_index, node_feats, W1, b1, W2, b2):
    real_epw = N_EDGES // NW
    pad = EPW - real_epw
    src = jnp.pad(edge_index[0].astype(jnp.int32).reshape(NW, real_epw),
                  ((0, 0), (0, pad))).reshape(-1)
    dst_fill = jnp.broadcast_to(
        N_NODES + jnp.arange(pad, dtype=jnp.int32) % (N_ACC - N_NODES),
        (NW, pad))
    dst = jnp.concatenate(
        [edge_index[1].astype(jnp.int32).reshape(NW, real_epw), dst_fill],
        axis=1).reshape(-1)
    h1 = _mm(node_feats, W1, b1)
    p = _edge_agg(src, dst, h1)
    h2 = _combine_mm(p[0], p[1], h1, W2, b2)
    q = _edge_agg(src, dst, h2)
    return _combine_relu(q[0], q[1], h2)


# speedup vs baseline: 1.7203x; 1.7203x over previous
import functools

import jax
import jax.numpy as jnp
from jax import lax
from jax.experimental import pallas as pl
from jax.experimental.pallas import tpu as pltpu
from jax.experimental.pallas import tpu_sc as plsc

N_NODES = 10000
N_EDGES = 320000
D = 128

NC = 2
NS = 16
NW = NC * NS
CHUNK = 80
NCH = 125
EPW = NCH * CHUNK
N_ACC = N_NODES
ROWS_PER_SUB = 624
TAIL_BASE = NS * ROWS_PER_SUB
TAIL = N_NODES - TAIL_BASE

_mesh = plsc.VectorSubcoreMesh(core_axis_name="c", subcore_axis_name="s")


@functools.partial(
    pl.kernel,
    mesh=_mesh,
    out_type=jax.ShapeDtypeStruct((2, N_NODES, D), jnp.float32),
    scratch_types=[
        pltpu.VMEM((EPW,), jnp.int32),
        pltpu.VMEM((EPW,), jnp.int32),
        pltpu.VMEM((CHUNK, D), jnp.float32),
        pltpu.VMEM((CHUNK, D), jnp.float32),
        pltpu.VMEM((CHUNK, D), jnp.float32),
        pltpu.VMEM_SHARED((N_ACC, D), jnp.float32),
        pltpu.SemaphoreType.DMA,
        pltpu.SemaphoreType.DMA,
        pltpu.SemaphoreType.DMA,
    ],
)
def _edge_agg(src_hbm, dst_hbm, h_hbm, out_hbm, sidx, didx, rows0, rows1,
              rows2, acc, semg0, semg1, semg2):
    cid = lax.axis_index("c")
    sid = lax.axis_index("s")
    wid = cid * NS + sid
    ebase = wid * EPW

    rowsb = (rows0, rows1, rows2)
    semg = (semg0, semg1, semg2)

    def _chunk(ref, i):
        return ref.at[pl.ds(pl.multiple_of(i * CHUNK, 8), CHUNK)]

    def _g_fire(i, b):
        pltpu.async_copy(h_hbm.at[_chunk(sidx, i)], rowsb[b], semg[b])

    def _g_wait(b):
        pltpu.make_async_copy(h_hbm.at[_chunk(sidx, 0)], rowsb[b],
                              semg[b]).wait()

    base = sid * ROWS_PER_SUB
    pltpu.async_copy(src_hbm.at[pl.ds(ebase, EPW)], sidx, semg0)
    pltpu.async_copy(dst_hbm.at[pl.ds(ebase, EPW)], didx, semg1)
    pltpu.async_copy(h_hbm.at[pl.ds(base, ROWS_PER_SUB)],
                     acc.at[pl.ds(base, ROWS_PER_SUB)], semg2)
    pltpu.make_async_copy(src_hbm.at[pl.ds(ebase, EPW)], sidx,
                          semg0).wait()
    _g_fire(0, 0)

    @pl.when(sid == NS - 1)
    def _():
        pltpu.sync_copy(h_hbm.at[pl.ds(TAIL_BASE, TAIL)],
                        acc.at[pl.ds(TAIL_BASE, TAIL)])

    pltpu.make_async_copy(dst_hbm.at[pl.ds(ebase, EPW)], didx,
                          semg1).wait()
    _g_fire(1, 1)
    pltpu.make_async_copy(h_hbm.at[pl.ds(base, ROWS_PER_SUB)],
                          acc.at[pl.ds(base, ROWS_PER_SUB)], semg2).wait()
    _g_fire(2, 2)

    plsc.subcore_barrier()

    def body(j, carry):
        c0 = 3 * j
        for k in range(3):
            i = c0 + k
            _g_wait(k)
            pltpu.sync_copy(rowsb[k], acc.at[_chunk(didx, i)], add=True)

            @pl.when(i + 3 < NCH)
            def _():
                _g_fire(i + 3, k)

        return carry

    lax.fori_loop(0, NCH // 3, body, 0)
    _g_wait(0)
    pltpu.sync_copy(rows0, acc.at[_chunk(didx, NCH - 2)], add=True)
    _g_wait(1)
    pltpu.sync_copy(rows1, acc.at[_chunk(didx, NCH - 1)], add=True)

    plsc.subcore_barrier()
    pltpu.sync_copy(acc.at[pl.ds(base, ROWS_PER_SUB)],
                    out_hbm.at[cid, pl.ds(base, ROWS_PER_SUB)])

    @pl.when(sid == NS - 1)
    def _():
        pltpu.sync_copy(acc.at[pl.ds(TAIL_BASE, TAIL)],
                        out_hbm.at[cid, pl.ds(TAIL_BASE, TAIL)])


_BLK = 1000
_GRID = N_NODES // _BLK


def _mm(x, W, b):
    def body(x_ref, w_ref, b_ref, o_ref):
        o_ref[...] = jnp.dot(x_ref[...], w_ref[...],
                             preferred_element_type=jnp.float32) + b_ref[...]

    return pl.pallas_call(
        body,
        grid=(_GRID,),
        in_specs=[pl.BlockSpec((_BLK, D), lambda i: (i, 0)),
                  pl.BlockSpec((D, D), lambda i: (0, 0)),
                  pl.BlockSpec((1, D), lambda i: (0, 0))],
        out_specs=pl.BlockSpec((_BLK, D), lambda i: (i, 0)),
        out_shape=jax.ShapeDtypeStruct((N_NODES, D), jnp.float32),
    )(x, W, b.reshape(1, D))


def _combine_mm(p0, p1, h, W, b):
    def body(p0_ref, p1_ref, h_ref, w_ref, b_ref, o_ref):
        z = jnp.maximum(p0_ref[...] + p1_ref[...] - h_ref[...], 0.0)
        o_ref[...] = jnp.dot(z, w_ref[...],
                             preferred_element_type=jnp.float32) + b_ref[...]

    return pl.pallas_call(
        body,
        grid=(_GRID,),
        in_specs=[pl.BlockSpec((_BLK, D), lambda i: (i, 0)),
                  pl.BlockSpec((_BLK, D), lambda i: (i, 0)),
                  pl.BlockSpec((_BLK, D), lambda i: (i, 0)),
                  pl.BlockSpec((D, D), lambda i: (0, 0)),
                  pl.BlockSpec((1, D), lambda i: (0, 0))],
        out_specs=pl.BlockSpec((_BLK, D), lambda i: (i, 0)),
        out_shape=jax.ShapeDtypeStruct((N_NODES, D), jnp.float32),
    )(p0, p1, h, W, b.reshape(1, D))


def _combine_relu(p0, p1, h):
    def body(p0_ref, p1_ref, h_ref, o_ref):
        o_ref[...] = jnp.maximum(p0_ref[...] + p1_ref[...] - h_ref[...], 0.0)

    return pl.pallas_call(
        body,
        grid=(_GRID,),
        in_specs=[pl.BlockSpec((_BLK, D), lambda i: (i, 0)),
                  pl.BlockSpec((_BLK, D), lambda i: (i, 0)),
                  pl.BlockSpec((_BLK, D), lambda i: (i, 0))],
        out_specs=pl.BlockSpec((_BLK, D), lambda i: (i, 0)),
        out_shape=jax.ShapeDtypeStruct((N_NODES, D), jnp.float32),
    )(p0, p1, h)


def kernel(edge_index, node_feats, W1, b1, W2, b2):
    src = edge_index[0].astype(jnp.int32)
    dst = edge_index[1].astype(jnp.int32)
    h1 = _mm(node_feats, W1, b1)
    p = _edge_agg(src, dst, h1)
    h2 = _combine_mm(p[0], p[1], h1, W2, b2)
    q = _edge_agg(src, dst, h2)
    return _combine_relu(q[0], q[1], h2)
